# baseline (speedup 1.0000x reference)
import functools

import jax
import jax.numpy as jnp
from jax import lax
from jax.experimental import pallas as pl
from jax.experimental.pallas import tpu as pltpu
from jax.experimental.pallas import tpu_sc as plsc

VOCAB = 1000
D = 128
L = 8
BATCH = 16384
B = BATCH * L

_info = plsc.get_sparse_core_info()
NC, NS, NLANES = _info.num_cores, _info.num_subcores, _info.num_lanes
NW = NC * NS
BPW = B // NW
R = 256
NCHUNK = BPW // R


def _body(tok_hbm, idx_hbm, pos_hbm, out_hbm, tok_sh, idx_v, buf0, buf1,
          pos_v, gsem0, gsem1, osem0, osem1, stage_sem):
    sid = lax.axis_index("s")
    wid = sid * NC + lax.axis_index("c")
    base = wid * BPW

    pltpu.sync_copy(idx_hbm.at[pl.ds(base, BPW)], idx_v)
    pltpu.sync_copy(pos_hbm, pos_v)

    @pl.when(sid == 0)
    def _stage_table():
        pltpu.async_copy(tok_hbm, tok_sh, stage_sem)

    bufs = (buf0, buf1)
    gsems = (gsem0, gsem1)
    osems = (osem0, osem1)

    def start_gather(g, b):
        src = tok_hbm if g < 2 else tok_sh
        return pltpu.async_copy(
            src.at[idx_v.at[pl.ds(g * R, R)]], bufs[b], gsems[b])

    def add_pos(buf):
        def j_body(j, _):
            sl = pl.ds(j * NLANES, NLANES)
            prow = [pos_v[l, sl] for l in range(L)]

            def grp_body(grp, _):
                row0 = grp * L
                for l in range(L):
                    buf[row0 + l, sl] = buf[row0 + l, sl] + prow[l]
                return 0

            lax.fori_loop(0, R // L, grp_body, 0)
            return 0

        lax.fori_loop(0, D // NLANES, j_body, 0)

    gcp = [None, None]
    scp = [None, None]
    gcp[0] = start_gather(0, 0)
    for g in range(NCHUNK):
        b = g & 1
        gcp[b].wait()
        if g + 1 < NCHUNK:
            if g + 1 == 2:
                @pl.when(sid == 0)
                def _wait_stage():
                    pltpu.make_async_copy(tok_hbm, tok_sh, stage_sem).wait()

                plsc.subcore_barrier()
            if scp[1 - b] is not None:
                scp[1 - b].wait()
            gcp[1 - b] = start_gather(g + 1, 1 - b)
        add_pos(bufs[b])
        scp[b] = pltpu.async_copy(
            bufs[b], out_hbm.at[pl.ds(base + g * R, R)], osems[b])
    scp[0].wait()
    scp[1].wait()


@functools.partial(jax.jit, static_argnums=())
def kernel(X, token_table, pos_table):
    idx = X.reshape(B)
    mesh = plsc.VectorSubcoreMesh(core_axis_name="c", subcore_axis_name="s")
    out = pl.kernel(
        _body,
        mesh=mesh,
        out_type=jax.ShapeDtypeStruct((B, D), jnp.float32),
        scratch_types=[
            pltpu.VMEM_SHARED((VOCAB, D), jnp.float32),
            pltpu.VMEM((BPW,), jnp.int32),
            pltpu.VMEM((R, D), jnp.float32),
            pltpu.VMEM((R, D), jnp.float32),
            pltpu.VMEM((L, D), jnp.float32),
            pltpu.SemaphoreType.DMA,
            pltpu.SemaphoreType.DMA,
            pltpu.SemaphoreType.DMA,
            pltpu.SemaphoreType.DMA,
            pltpu.SemaphoreType.DMA,
        ],
    )(token_table, idx, pos_table)
    return out.reshape(BATCH, L, D)


# speedup vs baseline: 1.0413x; 1.0413x over previous
import functools

import jax
import jax.numpy as jnp
from jax import lax
from jax.experimental import pallas as pl
from jax.experimental.pallas import tpu as pltpu
from jax.experimental.pallas import tpu_sc as plsc

VOCAB = 1000
D = 128
L = 8
BATCH = 16384
B = BATCH * L

_info = plsc.get_sparse_core_info()
NC, NS, NLANES = _info.num_cores, _info.num_subcores, _info.num_lanes
NW = NC * NS
BPW = B // NW
R = 256
NCHUNK = BPW // R


def _body(tok_hbm, idx_hbm, pos_hbm, out_hbm, tok_sh, idx_v, buf0, buf1,
          pos_v, gsem0, gsem1, osem0, osem1):
    sid = lax.axis_index("s")
    wid = sid * NC + lax.axis_index("c")
    base = wid * BPW

    @pl.when(sid == 0)
    def _stage_table():
        pltpu.sync_copy(tok_hbm, tok_sh)

    pltpu.sync_copy(idx_hbm.at[pl.ds(base, BPW)], idx_v)
    pltpu.sync_copy(pos_hbm, pos_v)
    plsc.subcore_barrier()

    bufs = (buf0, buf1)
    gsems = (gsem0, gsem1)
    osems = (osem0, osem1)

    def start_gather(g, b):
        return pltpu.async_copy(
            tok_sh.at[idx_v.at[pl.ds(g * R, R)]], bufs[b], gsems[b])

    def add_pos(buf):
        def j_body(j, _):
            sl = pl.ds(j * NLANES, NLANES)
            prow = [pos_v[l, sl] for l in range(L)]

            def grp_body(grp, _):
                row0 = grp * L
                for l in range(L):
                    buf[row0 + l, sl] = buf[row0 + l, sl] + prow[l]
                return 0

            lax.fori_loop(0, R // L, grp_body, 0)
            return 0

        lax.fori_loop(0, D // NLANES, j_body, 0)

    gcp = [None, None]
    scp = [None, None]
    gcp[0] = start_gather(0, 0)
    for g in range(NCHUNK):
        b = g & 1
        gcp[b].wait()
        if g + 1 < NCHUNK:
            if scp[1 - b] is not None:
                scp[1 - b].wait()
            gcp[1 - b] = start_gather(g + 1, 1 - b)
        add_pos(bufs[b])
        scp[b] = pltpu.async_copy(
            bufs[b], out_hbm.at[pl.ds(base + g * R, R)], osems[b])
    scp[0].wait()
    scp[1].wait()


@functools.partial(jax.jit, static_argnums=())
def kernel(X, token_table, pos_table):
    idx = X.reshape(B)
    mesh = plsc.VectorSubcoreMesh(core_axis_name="c", subcore_axis_name="s")
    out = pl.kernel(
        _body,
        mesh=mesh,
        out_type=jax.ShapeDtypeStruct((B, D), jnp.float32),
        scratch_types=[
            pltpu.VMEM_SHARED((VOCAB, D), jnp.float32),
            pltpu.VMEM((BPW,), jnp.int32),
            pltpu.VMEM((R, D), jnp.float32),
            pltpu.VMEM((R, D), jnp.float32),
            pltpu.VMEM((L, D), jnp.float32),
            pltpu.SemaphoreType.DMA,
            pltpu.SemaphoreType.DMA,
            pltpu.SemaphoreType.DMA,
            pltpu.SemaphoreType.DMA,
        ],
    )(token_table, idx, pos_table)
    return out.reshape(BATCH, L, D)
